# baseline (speedup 1.0000x reference)
import functools

import jax
import jax.numpy as jnp
import numpy as np
from jax.experimental import pallas as pl
from jax.experimental.pallas import tpu as pltpu

_BINS = 10
_ALPHA = 0.75
_EDGES = [np.float32(i) / np.float32(10.0) for i in range(_BINS + 1)]


def _ghm_body(x_ref, t_ref, out_ref, acc_ref, *, nsteps):
    step = pl.program_id(0)

    @pl.when(step == 0)
    def _init():
        acc_ref[...] = jnp.zeros_like(acc_ref)

    x = x_ref[...]
    t = t_ref[...]

    ax = jnp.abs(x)
    en = jnp.exp(-ax)
    loss = jnp.maximum(x, 0.0) - x * t + jnp.log1p(en)
    p1 = 1.0 / (1.0 + en)
    pred = jnp.where(x >= 0.0, p1, en * p1)
    g = jnp.abs(pred - t)
    f = g * jnp.float32(10.0)

    acc_ref[0:1, :] += jnp.sum(loss, axis=0, keepdims=True)
    for j in range(1, _BINS):
        acc_ref[j:j + 1, :] += jnp.sum(
            jnp.where(f >= jnp.float32(j), loss, 0.0), axis=0, keepdims=True)
    for i in range(1, _BINS + 1):
        acc_ref[15 + i:16 + i, :] += jnp.sum(
            (g >= _EDGES[i]).astype(jnp.float32), axis=0, keepdims=True)

    @pl.when(step == nsteps - 1)
    def _emit():
        a = acc_ref[...]
        r = a[:, 0:128]
        for k in range(1, 8):
            r = r + a[:, 128 * k:128 * (k + 1)]
        out_ref[...] = r


def kernel(x, target):
    n = x.size
    cols = 1024
    rows = n // cols
    r_blk = min(512, rows)
    grid = rows // r_blk

    xr = x.reshape(rows, cols)
    tr = target.reshape(rows, cols)

    out = pl.pallas_call(
        functools.partial(_ghm_body, nsteps=grid),
        grid=(grid,),
        in_specs=[
            pl.BlockSpec((r_blk, cols), lambda i: (i, 0)),
            pl.BlockSpec((r_blk, cols), lambda i: (i, 0)),
        ],
        out_specs=pl.BlockSpec((32, 128), lambda i: (0, 0)),
        out_shape=jax.ShapeDtypeStruct((32, 128), jnp.float32),
        scratch_shapes=[pltpu.VMEM((32, cols), jnp.float32)],
        compiler_params=pltpu.CompilerParams(
            dimension_semantics=("arbitrary",)),
    )(xr, tr)

    sums = jnp.sum(out, axis=1)
    t_j = sums[0:_BINS]
    c_i = sums[16:16 + _BINS]
    nf = jnp.float32(n)
    tot = jnp.concatenate([jnp.array([nf], jnp.float32), c_i[:-1]]) - c_i
    loss_sum = t_j - jnp.concatenate([t_j[1:], jnp.zeros((1,), jnp.float32)])
    w = jnp.clip(tot, 1.0, None) ** jnp.float32(-_ALPHA)
    return jnp.sum(loss_sum * w) / nf
